# speedup vs baseline: 1.8262x; 1.8262x over previous
import functools

import jax
import jax.numpy as jnp
from jax import lax
from jax.experimental import pallas as pl
from jax.experimental.pallas import tpu as pltpu
from jax.experimental.pallas import tpu_sc as plsc


def _build_gather(B, V, D, NC, NS):
    NW = NC * NS
    BPW = B // NW
    CHUNK = 8
    NCH = BPW // CHUNK
    NBUF = 2

    mesh = plsc.VectorSubcoreMesh(core_axis_name="c", subcore_axis_name="s")

    @functools.partial(
        pl.kernel,
        mesh=mesh,
        out_type=jax.ShapeDtypeStruct((B, D), jnp.float32),
        scratch_types=[
            pltpu.VMEM((BPW,), jnp.int32),
            pltpu.VMEM((CHUNK, D), jnp.float32),
            pltpu.VMEM((CHUNK, D), jnp.float32),
            pltpu.SemaphoreType.DMA,
            pltpu.SemaphoreType.DMA,
        ],
    )
    def k(table_hbm, ids_hbm, out_hbm, idx_v, buf0, buf1, sem0, sem1):
        wid = lax.axis_index("s") * NC + lax.axis_index("c")
        base = wid * BPW
        pltpu.sync_copy(ids_hbm.at[pl.ds(base, BPW)], idx_v)

        bufs = (buf0, buf1)
        sems = (sem0, sem1)

        def g_start(j, b):
            pltpu.async_copy(
                table_hbm.at[idx_v.at[pl.ds(j * CHUNK, CHUNK)]], bufs[b], sems[b]
            )

        def g_wait(j, b):
            pltpu.make_async_copy(
                table_hbm.at[idx_v.at[pl.ds(j * CHUNK, CHUNK)]], bufs[b], sems[b]
            ).wait()

        def s_out(j, b):
            pltpu.sync_copy(bufs[b], out_hbm.at[pl.ds(base + j * CHUNK, CHUNK)])

        for b in range(NBUF):
            g_start(b, b)

        def group(gi, carry):
            j0 = gi * NBUF
            for b in range(NBUF):
                j = j0 + b
                g_wait(j, b)
                s_out(j, b)
                g_start(j + NBUF, b)
            return carry

        lax.fori_loop(0, (NCH - NBUF) // NBUF, group, 0)

        for b in range(NBUF):
            j = NCH - NBUF + b
            g_wait(j, b)
            s_out(j, b)

    return k


def kernel(input_ids, embed_weight):
    V, D = embed_weight.shape
    B = input_ids.size
    info = plsc.get_sparse_core_info()
    ids_flat = input_ids.reshape(-1).astype(jnp.int32)
    gather = _build_gather(B, V, D, info.num_cores, info.num_subcores)
    out = gather(embed_weight, ids_flat)
    return out.reshape(*input_ids.shape, D)
